# baseline (speedup 1.0000x reference)
import functools

import jax
import jax.numpy as jnp
from jax import lax
from jax.experimental import pallas as pl
from jax.experimental.pallas import tpu as pltpu
from jax.experimental.pallas import tpu_sc as plsc

B = 64
L_IN = 1536
D_TISSUE = 64
D_TOKEN = 32
HIDDEN = 1024
H2 = HIDDEN // 2
K_SEQ = L_IN * D_TOKEN

NC = 2
NS = 16
NW = NC * NS
TOK = B * L_IN
TPW = TOK // NW
VOCAB = 65
VOCAB_W = VOCAB * D_TOKEN

KB = 2048
NK = K_SEQ // KB

_SQRT_HALF = 0.7071067811865476


def _gelu(x):
    return 0.5 * x * (1.0 + lax.erf(x * _SQRT_HALF))


def _sc_gather(seq_flat, idx_flat, tissue_table, tissue_id):
    mesh = plsc.VectorSubcoreMesh(core_axis_name="c", subcore_axis_name="s")

    @functools.partial(
        pl.kernel,
        out_type=(
            jax.ShapeDtypeStruct((TOK * D_TOKEN,), jnp.float32),
            jax.ShapeDtypeStruct((B, D_TISSUE), jnp.float32),
        ),
        mesh=mesh,
        scratch_types=(
            pltpu.VMEM((VOCAB_W,), jnp.float32),
            pltpu.VMEM((TPW,), jnp.int32),
            pltpu.VMEM((TPW * D_TOKEN,), jnp.float32),
            pltpu.VMEM((B,), jnp.int32),
            pltpu.VMEM((B, D_TISSUE), jnp.float32),
            pltpu.SemaphoreType.DMA,
        ),
        compiler_params=pltpu.CompilerParams(use_tc_tiling_on_sc=False,
                                             needs_layout_passes=False),
    )
    def body(seq_hbm, idx_hbm, ttab_hbm, tid_hbm, x_hbm, te_hbm,
             tablev, idxv, rowsv, tidv, trowsv, sem):
        wid = lax.axis_index("s") * NC + lax.axis_index("c")
        lane = lax.iota(jnp.int32, 16)
        pltpu.sync_copy(seq_hbm, tablev)
        pltpu.sync_copy(idx_hbm.at[pl.ds(wid * TPW, TPW)], idxv)

        def chunk(c, _):
            tok = idxv[pl.ds(c * 16, 16)]
            base = tok * D_TOKEN
            obase = c * (16 * D_TOKEN) + lane * D_TOKEN
            for d in range(D_TOKEN):
                v = plsc.load_gather(tablev, [base + d])
                plsc.store_scatter(rowsv, [obase + d], v)
            return _

        lax.fori_loop(0, TPW // 16, chunk, None)
        pltpu.sync_copy(rowsv, x_hbm.at[pl.ds(wid * (TPW * D_TOKEN), TPW * D_TOKEN)])

        @pl.when(wid == 0)
        def _():
            pltpu.sync_copy(tid_hbm, tidv)
            pltpu.async_copy(ttab_hbm.at[tidv], trowsv, sem).wait()
            pltpu.sync_copy(trowsv, te_hbm)

    return body(seq_flat, idx_flat, tissue_table, tissue_id)


def _mlp(x2d, te, b1r, W2, b2r, W3, b3r, W1):
    def body(x_ref, te_ref, b1_ref, w2_ref, b2_ref, w3_ref, b3_ref, w1_hbm,
             y_ref, w1buf, w1t, acc, sems, semt):
        k = pl.program_id(0)

        def w1_copy(kk, slot):
            return pltpu.make_async_copy(
                w1_hbm.at[pl.ds(D_TISSUE + kk * KB, KB), :],
                w1buf.at[slot], sems.at[slot])

        @pl.when(k == 0)
        def _():
            pltpu.make_async_copy(w1_hbm.at[pl.ds(0, D_TISSUE), :], w1t, semt).start()
            w1_copy(0, 0).start()

        @pl.when(k + 1 < NK)
        def _():
            w1_copy(k + 1, (k + 1) % 2).start()

        @pl.when(k == 0)
        def _():
            pltpu.make_async_copy(w1_hbm.at[pl.ds(0, D_TISSUE), :], w1t, semt).wait()
            acc[...] = (
                jnp.dot(te_ref[...], w1t[...], preferred_element_type=jnp.float32)
                + b1_ref[...])

        w1_copy(k, k % 2).wait()
        acc[...] += jnp.dot(x_ref[...], w1buf[k % 2],
                            preferred_element_type=jnp.float32)

        @pl.when(k == NK - 1)
        def _():
            h = _gelu(acc[...])
            h2 = _gelu(jnp.dot(h, w2_ref[...], preferred_element_type=jnp.float32)
                       + b2_ref[...])
            y_ref[...] = (
                jnp.dot(h2, w3_ref[...], preferred_element_type=jnp.float32)
                + b3_ref[...])

    return pl.pallas_call(
        body,
        grid=(NK,),
        in_specs=[
            pl.BlockSpec((B, KB), lambda k: (0, k)),
            pl.BlockSpec((B, D_TISSUE), lambda k: (0, 0)),
            pl.BlockSpec((1, HIDDEN), lambda k: (0, 0)),
            pl.BlockSpec((HIDDEN, H2), lambda k: (0, 0)),
            pl.BlockSpec((1, H2), lambda k: (0, 0)),
            pl.BlockSpec((H2, 1), lambda k: (0, 0)),
            pl.BlockSpec((1, 1), lambda k: (0, 0)),
            pl.BlockSpec(memory_space=pl.ANY),
        ],
        out_specs=pl.BlockSpec((B, 1), lambda k: (0, 0)),
        out_shape=jax.ShapeDtypeStruct((B, 1), jnp.float32),
        scratch_shapes=[
            pltpu.VMEM((2, KB, HIDDEN), jnp.float32),
            pltpu.VMEM((D_TISSUE, HIDDEN), jnp.float32),
            pltpu.VMEM((B, HIDDEN), jnp.float32),
            pltpu.SemaphoreType.DMA((2,)),
            pltpu.SemaphoreType.DMA,
        ],
        compiler_params=pltpu.CompilerParams(
            dimension_semantics=("arbitrary",)),
    )(x2d, te, b1r, W2, b2r, W3, b3r, W1)


def kernel(rna_data, tissue_id, tissue_table, seq_table, W1, b1, W2, b2, W3, b3):
    xf, te = _sc_gather(seq_table.reshape(VOCAB_W), rna_data.reshape(TOK),
                        tissue_table, tissue_id)
    x2d = xf.reshape(B, K_SEQ)
    return _mlp(x2d, te, b1.reshape(1, HIDDEN), W2, b2.reshape(1, H2),
                W3, b3.reshape(1, 1), W1)


# speedup vs baseline: 1.3236x; 1.3236x over previous
import functools

import jax
import jax.numpy as jnp
from jax import lax
from jax.experimental import pallas as pl
from jax.experimental.pallas import tpu as pltpu
from jax.experimental.pallas import tpu_sc as plsc

B = 64
L_IN = 1536
D_TISSUE = 64
D_TOKEN = 32
HIDDEN = 1024
H2 = HIDDEN // 2
K_SEQ = L_IN * D_TOKEN

NC = 2
NS = 16
NW = NC * NS
TOK = B * L_IN
TPW = TOK // NW
VOCAB = 65
CHUNK = 128
NCH = TPW // CHUNK

KB = 2048
NK = K_SEQ // KB

_SQRT_HALF = 0.7071067811865476


def _gelu(x):
    return 0.5 * x * (1.0 + lax.erf(x * _SQRT_HALF))


def _sc_gather(seq_table, idx2d, tissue_table, tissue_id):
    mesh = plsc.VectorSubcoreMesh(core_axis_name="c", subcore_axis_name="s")

    @functools.partial(
        pl.kernel,
        out_type=(
            jax.ShapeDtypeStruct((TOK, D_TOKEN), jnp.float32),
            jax.ShapeDtypeStruct((B, D_TISSUE), jnp.float32),
        ),
        mesh=mesh,
        scratch_types=(
            pltpu.VMEM((TPW,), jnp.int32),
            pltpu.VMEM((TPW, D_TOKEN), jnp.float32),
            pltpu.VMEM((B,), jnp.int32),
            pltpu.VMEM((B, D_TISSUE), jnp.float32),
            pltpu.SemaphoreType.DMA,
        ),
        compiler_params=pltpu.CompilerParams(use_tc_tiling_on_sc=False),
    )
    def body(seq_hbm, idx_hbm, ttab_hbm, tid_hbm, x_hbm, te_hbm,
             idxv, rowsv, tidv, trowsv, sem):
        wid = lax.axis_index("s") * NC + lax.axis_index("c")
        pltpu.sync_copy(idx_hbm.at[pl.ds(wid * TPW, TPW)], idxv)
        pltpu.async_copy(seq_hbm.at[idxv], rowsv, sem).wait()
        pltpu.sync_copy(rowsv, x_hbm.at[pl.ds(wid * TPW, TPW)])

        @pl.when(wid == 0)
        def _():
            pltpu.sync_copy(tid_hbm, tidv)
            pltpu.async_copy(ttab_hbm.at[tidv], trowsv, sem).wait()
            pltpu.sync_copy(trowsv, te_hbm)

    return body(seq_table, idx2d, tissue_table, tissue_id)


def _mlp(x2d, te, b1r, W2, b2r, W3, b3r, W1):
    def body(x_ref, te_ref, b1_ref, w2_ref, b2_ref, w3_ref, b3_ref, w1_hbm,
             y_ref, w1buf, w1t, acc, sems, semt):
        k = pl.program_id(0)

        def w1_copy(kk, slot):
            return pltpu.make_async_copy(
                w1_hbm.at[pl.ds(D_TISSUE + kk * KB, KB), :],
                w1buf.at[slot], sems.at[slot])

        @pl.when(k == 0)
        def _():
            pltpu.make_async_copy(w1_hbm.at[pl.ds(0, D_TISSUE), :], w1t, semt).start()
            w1_copy(0, 0).start()

        @pl.when(k + 1 < NK)
        def _():
            w1_copy(k + 1, (k + 1) % 2).start()

        @pl.when(k == 0)
        def _():
            pltpu.make_async_copy(w1_hbm.at[pl.ds(0, D_TISSUE), :], w1t, semt).wait()
            acc[...] = (
                jnp.dot(te_ref[...], w1t[...], preferred_element_type=jnp.float32)
                + b1_ref[...])

        w1_copy(k, k % 2).wait()
        acc[...] += jnp.dot(x_ref[...], w1buf[k % 2],
                            preferred_element_type=jnp.float32)

        @pl.when(k == NK - 1)
        def _():
            h = _gelu(acc[...])
            h2 = _gelu(jnp.dot(h, w2_ref[...], preferred_element_type=jnp.float32)
                       + b2_ref[...])
            y_ref[...] = (
                jnp.dot(h2, w3_ref[...], preferred_element_type=jnp.float32)
                + b3_ref[...])

    return pl.pallas_call(
        body,
        grid=(NK,),
        in_specs=[
            pl.BlockSpec((B, KB), lambda k: (0, k)),
            pl.BlockSpec((B, D_TISSUE), lambda k: (0, 0)),
            pl.BlockSpec((1, HIDDEN), lambda k: (0, 0)),
            pl.BlockSpec((HIDDEN, H2), lambda k: (0, 0)),
            pl.BlockSpec((1, H2), lambda k: (0, 0)),
            pl.BlockSpec((H2, 1), lambda k: (0, 0)),
            pl.BlockSpec((1, 1), lambda k: (0, 0)),
            pl.BlockSpec(memory_space=pl.ANY),
        ],
        out_specs=pl.BlockSpec((B, 1), lambda k: (0, 0)),
        out_shape=jax.ShapeDtypeStruct((B, 1), jnp.float32),
        scratch_shapes=[
            pltpu.VMEM((2, KB, HIDDEN), jnp.float32),
            pltpu.VMEM((D_TISSUE, HIDDEN), jnp.float32),
            pltpu.VMEM((B, HIDDEN), jnp.float32),
            pltpu.SemaphoreType.DMA((2,)),
            pltpu.SemaphoreType.DMA,
        ],
        compiler_params=pltpu.CompilerParams(
            dimension_semantics=("arbitrary",)),
    )(x2d, te, b1r, W2, b2r, W3, b3r, W1)


def kernel(rna_data, tissue_id, tissue_table, seq_table, W1, b1, W2, b2, W3, b3):
    x3, te = _sc_gather(seq_table, rna_data.reshape(TOK), tissue_table,
                        tissue_id)
    x2d = x3.reshape(B, K_SEQ)
    return _mlp(x2d, te, b1.reshape(1, HIDDEN), W2, b2.reshape(1, H2),
                W3, b3.reshape(1, 1), W1)
